# baseline (speedup 1.0000x reference)
import functools
import math

import jax
import jax.numpy as jnp
from jax import lax
from jax.experimental import pallas as pl
from jax.experimental.pallas import tpu as pltpu

_B, _CD, _H, _W = 2, 192, 224, 224
_HEADS = 8
_CH = _CD // _HEADS
_HH, _WW = _H // 2, _W // 2
_N = _HH * _WW
_NT = _N // 7
_KKS = (12, 16, 18, 19)


_RS = 32


def _qkv_pool_kernel(x_ref, w_ref, out_ref):
    w = w_ref[...]
    ii = lax.broadcasted_iota(jnp.int32, (_W, _WW), 0)
    jj = lax.broadcasted_iota(jnp.int32, (_W, _WW), 1)
    sel_even = (ii == 2 * jj).astype(jnp.float32)
    sel_odd = (ii == 2 * jj + 1).astype(jnp.float32)
    for rp in range(_RS // 2):
        y0 = jnp.dot(w, x_ref[0, :, 2 * rp, :],
                     preferred_element_type=jnp.float32)
        y1 = jnp.dot(w, x_ref[0, :, 2 * rp + 1, :],
                     preferred_element_type=jnp.float32)
        yr = jnp.maximum(y0, y1)
        yc = jnp.maximum(
            jnp.dot(yr, sel_even, preferred_element_type=jnp.float32),
            jnp.dot(yr, sel_odd, preferred_element_type=jnp.float32))
        out_ref[0, :, rp, :] = yc


def _dw_kernel(p_ref, w_ref, out_ref):
    x = p_ref[0]
    cb = x.shape[0]

    def shift_rows(t, dy):
        z = jnp.zeros((cb, 1, _WW), jnp.float32)
        if dy == 1:
            return jnp.concatenate([t[:, 1:, :], z], axis=1)
        return jnp.concatenate([z, t[:, :-1, :]], axis=1)

    def shift_cols(t, dx):
        z = jnp.zeros((cb, _HH, 1), jnp.float32)
        if dx == 1:
            return jnp.concatenate([t[:, :, 1:], z], axis=2)
        return jnp.concatenate([z, t[:, :, :-1]], axis=2)

    cols = (shift_cols(x, -1), x, shift_cols(x, 1))

    def row_tap(ky):
        t = cols[0] * w_ref[:, ky, 0].reshape(cb, 1, 1)
        t = t + cols[1] * w_ref[:, ky, 1].reshape(cb, 1, 1)
        t = t + cols[2] * w_ref[:, ky, 2].reshape(cb, 1, 1)
        return t

    acc = row_tap(1)
    acc = acc + shift_rows(row_tap(0), -1)
    acc = acc + shift_rows(row_tap(2), 1)
    out_ref[0] = acc


def _compute_abd(q, k, t_ref, av_ref, out):
    gram = lax.dot_general(q, k, (((1,), (1,)), ((), ())),
                           preferred_element_type=jnp.float32)
    qss = jnp.sum(q * q, axis=1, keepdims=True)
    kss = jnp.sum(k * k, axis=1, keepdims=True)
    invq = 1.0 / jnp.maximum(jnp.sqrt(qss), 1e-12)
    invk = 1.0 / jnp.maximum(jnp.sqrt(kss), 1e-12)
    scaled = gram * invq * invk.reshape(1, _CD) * t_ref[...]
    av = av_ref[...]
    out[...] = jnp.zeros((_CD, _CD), jnp.float32)
    for h in range(_HEADS):
        c0 = h * _CH
        a = scaled[c0:c0 + _CH, c0:c0 + _CH]
        ae = a[:, None, :]
        ad = a[:, :, None]
        gt = jnp.sum((ae > ad).astype(jnp.float32), axis=2)
        ie = lax.broadcasted_iota(jnp.int32, (_CH, _CH, _CH), 2)
        idd = lax.broadcasted_iota(jnp.int32, (_CH, _CH, _CH), 1)
        eq = jnp.sum(((ae == ad) & (ie < idd)).astype(jnp.float32), axis=2)
        rank = gt + eq
        rowmax = jnp.max(a, axis=1, keepdims=True)
        e = jnp.exp(a - rowmax)
        comb = jnp.zeros((_CH, _CH), jnp.float32)
        for i, kk in enumerate(_KKS):
            m = (rank < kk).astype(jnp.float32)
            em = e * m
            s = jnp.sum(em, axis=1, keepdims=True)
            comb = comb + em * (av[i:i + 1, :] / s)
        out[c0:c0 + _CH, c0:c0 + _CH] = comb


_RU = 8


def _out_kernel(q_ref, k_ref, t_ref, av_ref, v_ref, pw_ref, out_ref,
                abd_ref):
    @pl.when(pl.program_id(1) == 0)
    def _():
        _compute_abd(q_ref[0], k_ref[0], t_ref, av_ref, abd_ref)

    abd = abd_ref[...]
    pw = pw_ref[...]
    ii = lax.broadcasted_iota(jnp.int32, (_WW, _W), 0)
    jj = lax.broadcasted_iota(jnp.int32, (_WW, _W), 1)
    dup = (ii == jj // 2).astype(jnp.float32)
    for r in range(_RU):
        t = jnp.dot(abd, v_ref[0, :, r, :],
                    preferred_element_type=jnp.float32)
        g = 0.5 * t * (1.0 + lax.erf(t * (1.0 / math.sqrt(2.0))))
        y = jnp.dot(pw, g, preferred_element_type=jnp.float32)
        row = jnp.dot(y, dup, preferred_element_type=jnp.float32)
        out_ref[0, :, 2 * r, :] = row
        out_ref[0, :, 2 * r + 1, :] = row


@functools.partial(jax.jit, static_argnums=())
def kernel(x, temperature, qkv_w, dw_w, proj_w, a1, a2, a3, a4):
    b, c, h, w = x.shape
    f32 = jnp.float32

    wqkv = qkv_w[:, :, 0, 0]
    wdw = dw_w[:, 0, :, :]
    wproj = proj_w[:, :, 0, 0]
    t192 = jnp.repeat(temperature[:, 0, 0], _CH).reshape(_CD, 1).astype(f32)
    av = jnp.concatenate([a1, a2, a3, a4]).reshape(4, 1).astype(f32)

    pooled = pl.pallas_call(
        _qkv_pool_kernel,
        grid=(b, _H // _RS),
        in_specs=[
            pl.BlockSpec((1, c, _RS, _W), lambda i, j: (i, 0, j, 0)),
            pl.BlockSpec((3 * c, c), lambda i, j: (0, 0)),
        ],
        out_specs=pl.BlockSpec((1, 3 * c, _RS // 2, _WW),
                               lambda i, j: (i, 0, j, 0)),
        out_shape=jax.ShapeDtypeStruct((b, 3 * c, _HH, _WW), f32),
    )(x, wqkv)
    pooled = pooled.reshape(b, 3 * c, _N)

    cb = 96
    dw = pl.pallas_call(
        _dw_kernel,
        grid=(b, 3 * c // cb),
        in_specs=[
            pl.BlockSpec((1, cb, _HH, _WW), lambda i, j: (i, j, 0, 0)),
            pl.BlockSpec((cb, 3, 3), lambda i, j: (j, 0, 0)),
        ],
        out_specs=pl.BlockSpec((1, cb, _HH, _WW), lambda i, j: (i, j, 0, 0)),
        out_shape=jax.ShapeDtypeStruct((b, 3 * c, _HH, _WW), f32),
    )(pooled.reshape(b, 3 * c, _HH, _WW), wdw)
    dwf = dw.reshape(b, 3 * c, _N)

    up = pl.pallas_call(
        _out_kernel,
        grid=(b, _HH // _RU),
        in_specs=[
            pl.BlockSpec((1, c, _N), lambda i, j: (i, 0, 0)),
            pl.BlockSpec((1, c, _N), lambda i, j: (i, 1, 0)),
            pl.BlockSpec((c, 1), lambda i, j: (0, 0)),
            pl.BlockSpec((4, 1), lambda i, j: (0, 0)),
            pl.BlockSpec((1, c, _RU, _WW), lambda i, j: (i, 2, j, 0)),
            pl.BlockSpec((c, c), lambda i, j: (0, 0)),
        ],
        out_specs=pl.BlockSpec((1, c, 2 * _RU, _W), lambda i, j: (i, 0, j, 0)),
        out_shape=jax.ShapeDtypeStruct((b, c, _H, _W), f32),
        scratch_shapes=[pltpu.VMEM((c, c), f32)],
    )(dwf, dwf, t192, av, dw, wproj)
    return up


# speedup vs baseline: 1.0161x; 1.0161x over previous
import functools
import math

import jax
import jax.numpy as jnp
from jax import lax
from jax.experimental import pallas as pl
from jax.experimental.pallas import tpu as pltpu

_B, _CD, _H, _W = 2, 192, 224, 224
_HEADS = 8
_CH = _CD // _HEADS
_HH, _WW = _H // 2, _W // 2
_N = _HH * _WW
_NT = _N // 7
_KKS = (12, 16, 18, 19)


_RS = 32


def _qkv_pool_kernel(x_ref, w_ref, out_ref):
    w = w_ref[...]
    ii = lax.broadcasted_iota(jnp.int32, (_W, _WW), 0)
    jj = lax.broadcasted_iota(jnp.int32, (_W, _WW), 1)
    sel_even = (ii == 2 * jj).astype(jnp.float32)
    sel_odd = (ii == 2 * jj + 1).astype(jnp.float32)
    for rp in range(_RS // 2):
        y0 = jnp.dot(w, x_ref[0, :, 2 * rp, :],
                     preferred_element_type=jnp.float32)
        y1 = jnp.dot(w, x_ref[0, :, 2 * rp + 1, :],
                     preferred_element_type=jnp.float32)
        yr = jnp.maximum(y0, y1)
        yc = jnp.maximum(
            jnp.dot(yr, sel_even, preferred_element_type=jnp.float32),
            jnp.dot(yr, sel_odd, preferred_element_type=jnp.float32))
        out_ref[0, :, rp, :] = yc


def _dw_kernel(p_ref, w_ref, out_ref):
    x = p_ref[0]
    cb = x.shape[0]

    def shift_rows(t, dy):
        z = jnp.zeros((cb, 1, _WW), jnp.float32)
        if dy == 1:
            return jnp.concatenate([t[:, 1:, :], z], axis=1)
        return jnp.concatenate([z, t[:, :-1, :]], axis=1)

    def shift_cols(t, dx):
        z = jnp.zeros((cb, _HH, 1), jnp.float32)
        if dx == 1:
            return jnp.concatenate([t[:, :, 1:], z], axis=2)
        return jnp.concatenate([z, t[:, :, :-1]], axis=2)

    cols = (shift_cols(x, -1), x, shift_cols(x, 1))

    def row_tap(ky):
        t = cols[0] * w_ref[:, ky, 0].reshape(cb, 1, 1)
        t = t + cols[1] * w_ref[:, ky, 1].reshape(cb, 1, 1)
        t = t + cols[2] * w_ref[:, ky, 2].reshape(cb, 1, 1)
        return t

    acc = row_tap(1)
    acc = acc + shift_rows(row_tap(0), -1)
    acc = acc + shift_rows(row_tap(2), 1)
    out_ref[0] = acc


def _attn_kernel(q_ref, k_ref, t_ref, av_ref, out_ref):
    q = q_ref[0]
    k = k_ref[0]
    gram = lax.dot_general(q, k, (((1,), (1,)), ((), ())),
                           preferred_element_type=jnp.float32)
    qss = jnp.sum(q * q, axis=1, keepdims=True)
    kss = jnp.sum(k * k, axis=1, keepdims=True)
    invq = 1.0 / jnp.maximum(jnp.sqrt(qss), 1e-12)
    invk = 1.0 / jnp.maximum(jnp.sqrt(kss), 1e-12)
    scaled = gram * invq * invk.reshape(1, _CD) * t_ref[...]
    av = av_ref[...]
    out_ref[0] = jnp.zeros((_CD, _CD), jnp.float32)
    for h in range(_HEADS):
        c0 = h * _CH
        a = scaled[c0:c0 + _CH, c0:c0 + _CH]
        ae = a[:, None, :]
        ad = a[:, :, None]
        gt = jnp.sum((ae > ad).astype(jnp.float32), axis=2)
        ie = lax.broadcasted_iota(jnp.int32, (_CH, _CH, _CH), 2)
        idd = lax.broadcasted_iota(jnp.int32, (_CH, _CH, _CH), 1)
        eq = jnp.sum(((ae == ad) & (ie < idd)).astype(jnp.float32), axis=2)
        rank = gt + eq
        rowmax = jnp.max(a, axis=1, keepdims=True)
        e = jnp.exp(a - rowmax)
        comb = jnp.zeros((_CH, _CH), jnp.float32)
        for i, kk in enumerate(_KKS):
            m = (rank < kk).astype(jnp.float32)
            em = e * m
            s = jnp.sum(em, axis=1, keepdims=True)
            comb = comb + em * (av[i:i + 1, :] / s)
        out_ref[0, c0:c0 + _CH, c0:c0 + _CH] = comb


_RU = 16


def _out_kernel(abd_ref, v_ref, pw_ref, out_ref):
    abd = abd_ref[0]
    pw = pw_ref[...]
    ii = lax.broadcasted_iota(jnp.int32, (_WW, _W), 0)
    jj = lax.broadcasted_iota(jnp.int32, (_WW, _W), 1)
    dup = (ii == jj // 2).astype(jnp.float32)
    for r in range(_RU):
        t = jnp.dot(abd, v_ref[0, :, r, :],
                    preferred_element_type=jnp.float32)
        g = 0.5 * t * (1.0 + lax.erf(t * (1.0 / math.sqrt(2.0))))
        y = jnp.dot(pw, g, preferred_element_type=jnp.float32)
        row = jnp.dot(y, dup, preferred_element_type=jnp.float32)
        out_ref[0, :, 2 * r, :] = row
        out_ref[0, :, 2 * r + 1, :] = row


@functools.partial(jax.jit, static_argnums=())
def kernel(x, temperature, qkv_w, dw_w, proj_w, a1, a2, a3, a4):
    b, c, h, w = x.shape
    f32 = jnp.float32

    wqkv = qkv_w[:, :, 0, 0]
    wdw = dw_w[:, 0, :, :]
    wproj = proj_w[:, :, 0, 0]
    t192 = jnp.repeat(temperature[:, 0, 0], _CH).reshape(_CD, 1).astype(f32)
    av = jnp.concatenate([a1, a2, a3, a4]).reshape(4, 1).astype(f32)

    pooled = pl.pallas_call(
        _qkv_pool_kernel,
        grid=(b, _H // _RS),
        in_specs=[
            pl.BlockSpec((1, c, _RS, _W), lambda i, j: (i, 0, j, 0)),
            pl.BlockSpec((3 * c, c), lambda i, j: (0, 0)),
        ],
        out_specs=pl.BlockSpec((1, 3 * c, _RS // 2, _WW),
                               lambda i, j: (i, 0, j, 0)),
        out_shape=jax.ShapeDtypeStruct((b, 3 * c, _HH, _WW), f32),
    )(x, wqkv)
    pooled = pooled.reshape(b, 3 * c, _N)

    cb = 96
    dw = pl.pallas_call(
        _dw_kernel,
        grid=(b, 3 * c // cb),
        in_specs=[
            pl.BlockSpec((1, cb, _HH, _WW), lambda i, j: (i, j, 0, 0)),
            pl.BlockSpec((cb, 3, 3), lambda i, j: (j, 0, 0)),
        ],
        out_specs=pl.BlockSpec((1, cb, _HH, _WW), lambda i, j: (i, j, 0, 0)),
        out_shape=jax.ShapeDtypeStruct((b, 3 * c, _HH, _WW), f32),
    )(pooled.reshape(b, 3 * c, _HH, _WW), wdw)
    dwf = dw.reshape(b, 3 * c, _N)

    abd = pl.pallas_call(
        _attn_kernel,
        grid=(b,),
        in_specs=[
            pl.BlockSpec((1, c, _N), lambda i: (i, 0, 0)),
            pl.BlockSpec((1, c, _N), lambda i: (i, 1, 0)),
            pl.BlockSpec((c, 1), lambda i: (0, 0)),
            pl.BlockSpec((4, 1), lambda i: (0, 0)),
        ],
        out_specs=pl.BlockSpec((1, c, c), lambda i: (i, 0, 0)),
        out_shape=jax.ShapeDtypeStruct((b, c, c), f32),
    )(dwf, dwf, t192, av)

    up = pl.pallas_call(
        _out_kernel,
        grid=(b, _HH // _RU),
        in_specs=[
            pl.BlockSpec((1, c, c), lambda i, j: (i, 0, 0)),
            pl.BlockSpec((1, c, _RU, _WW), lambda i, j: (i, 2, j, 0)),
            pl.BlockSpec((c, c), lambda i, j: (0, 0)),
        ],
        out_specs=pl.BlockSpec((1, c, 2 * _RU, _W), lambda i, j: (i, 0, j, 0)),
        out_shape=jax.ShapeDtypeStruct((b, c, _H, _W), f32),
    )(abd, dw, wproj)
    return up
